# baseline (speedup 1.0000x reference)
import functools
import math

import jax
import jax.numpy as jnp
from jax import lax
from jax.experimental import pallas as pl
from jax.experimental.pallas import tpu as pltpu
from jax.experimental.pallas import tpu_sc as plsc

_STD = (0.1, 0.1, 0.2, 0.2)
_MAX_RATIO = abs(math.log(0.016))


def _tc_body(bn, n, num_classes, lg_ref, rg_ref, pr_ref,
             o0_ref, o1_ref, o2_ref, o3_ref, pcum_ref, valid_ref, nv_ref,
             carry):
    c = pl.program_id(0)

    @pl.when(c == 0)
    def _():
        carry[0] = 0

    lg = lg_ref[...]
    mx = jnp.max(lg, axis=0, keepdims=True)
    rowid = lax.broadcasted_iota(jnp.int32, lg.shape, 0)
    am = jnp.min(jnp.where(lg == mx, rowid, num_classes), axis=0,
                 keepdims=True)

    rg = rg_ref[...]
    sel = lax.broadcasted_iota(jnp.int32, (num_classes, 1, bn), 0) == am
    delta = jnp.sum(jnp.where(sel, rg, 0.0), axis=0)

    dx = delta[0:1] * _STD[0]
    dy = delta[1:2] * _STD[1]
    dw = jnp.clip(delta[2:3] * _STD[2], -_MAX_RATIO, _MAX_RATIO)
    dh = jnp.clip(delta[3:4] * _STD[3], -_MAX_RATIO, _MAX_RATIO)

    pr = pr_ref[...]
    px1, py1, px2, py2 = pr[0:1], pr[1:2], pr[2:3], pr[3:4]
    px = (px1 + px2) * 0.5
    py = (py1 + py2) * 0.5
    pw = px2 - px1
    ph = py2 - py1
    gw = pw * jnp.exp(dw)
    gh = ph * jnp.exp(dh)
    gx = px + pw * dx
    gy = py + ph * dy
    o0 = jnp.clip(gx - gw * 0.5, 0.0, 1.0)
    o1 = jnp.clip(gy - gh * 0.5, 0.0, 1.0)
    o2 = jnp.clip(gx + gw * 0.5, 0.0, 1.0)
    o3 = jnp.clip(gy + gh * 0.5, 0.0, 1.0)

    lane = lax.broadcasted_iota(jnp.int32, (1, bn), 1) + c * bn
    vmask = (jnp.max(pr, axis=0, keepdims=True) > 0.0) & (lane < n)
    zf = jnp.zeros((1, bn), jnp.float32)
    o0 = jnp.where(vmask, o0, zf)
    o1 = jnp.where(vmask, o1, zf)
    o2 = jnp.where(vmask, o2, zf)
    o3 = jnp.where(vmask, o3, zf)

    valid = vmask.astype(jnp.int32)
    x = valid
    s = 1
    while s < bn:
        x = x + jnp.concatenate(
            [jnp.zeros((1, s), jnp.int32), x[:, : bn - s]], axis=1)
        s *= 2
    base = carry[0]
    tot = base + jnp.sum(valid)
    carry[0] = tot

    o0_ref[...] = o0.reshape(bn)
    o1_ref[...] = o1.reshape(bn)
    o2_ref[...] = o2.reshape(bn)
    o3_ref[...] = o3.reshape(bn)
    pcum_ref[...] = ((x - valid) + base).reshape(bn)
    valid_ref[...] = valid.reshape(bn)
    nv_ref[...] = jnp.broadcast_to(tot, (16,)).astype(jnp.int32)


def _tc_stage(logits_t, regress_t, props_t, np_rows):
    cnum = logits_t.shape[0]
    n = logits_t.shape[2]
    bn = 4096
    grid = np_rows // bn
    body = functools.partial(_tc_body, bn, n, cnum)
    o1d = jax.ShapeDtypeStruct((np_rows,), jnp.float32)
    i1d = jax.ShapeDtypeStruct((np_rows,), jnp.int32)
    return pl.pallas_call(
        body,
        grid=(grid,),
        in_specs=[
            pl.BlockSpec((cnum, 1, bn), lambda i: (0, 0, i)),
            pl.BlockSpec((cnum, 4, bn), lambda i: (0, 0, i)),
            pl.BlockSpec((4, bn), lambda i: (0, i)),
        ],
        out_specs=[
            pl.BlockSpec((bn,), lambda i: (i,)),
            pl.BlockSpec((bn,), lambda i: (i,)),
            pl.BlockSpec((bn,), lambda i: (i,)),
            pl.BlockSpec((bn,), lambda i: (i,)),
            pl.BlockSpec((bn,), lambda i: (i,)),
            pl.BlockSpec((bn,), lambda i: (i,)),
            pl.BlockSpec((16,), lambda i: (0,)),
        ],
        out_shape=[o1d, o1d, o1d, o1d, i1d, i1d,
                   jax.ShapeDtypeStruct((16,), jnp.int32)],
        scratch_shapes=[pltpu.SMEM((1,), jnp.int32)],
    )(logits_t, regress_t, props_t)


def _sc_stage(o0, o1, o2, o3, pcum, valid, nv16, np_rows, nw, nc):
    bt = np_rows // nw
    nch = bt // 128
    ng = bt // 16
    mesh = plsc.VectorSubcoreMesh(core_axis_name="c", subcore_axis_name="s")

    @functools.partial(
        pl.kernel,
        mesh=mesh,
        compiler_params=pltpu.CompilerParams(
            needs_layout_passes=False, use_tc_tiling_on_sc=False),
        out_type=jax.ShapeDtypeStruct((np_rows, 8), jnp.float32),
        scratch_types=[
            pltpu.VMEM((bt,), jnp.float32),
            pltpu.VMEM((bt,), jnp.float32),
            pltpu.VMEM((bt,), jnp.float32),
            pltpu.VMEM((bt,), jnp.float32),
            pltpu.VMEM((bt,), jnp.int32),
            pltpu.VMEM((bt,), jnp.int32),
            pltpu.VMEM((bt, 8), jnp.float32),
            pltpu.VMEM((nch, 128), jnp.int32),
            pltpu.VMEM((16,), jnp.int32),
            pltpu.SemaphoreType.DMA,
        ],
    )
    def sc_kernel(o0_hbm, o1_hbm, o2_hbm, o3_hbm, pcum_hbm, valid_hbm,
                  nv_hbm, out_hbm, o0_v, o1_v, o2_v, o3_v, pcum_v, valid_v,
                  rows_v, tgt_v, nv_v, sem):
        wid = lax.axis_index("s") * nc + lax.axis_index("c")
        base = wid * bt
        pltpu.sync_copy(o0_hbm.at[pl.ds(base, bt)], o0_v)
        pltpu.sync_copy(o1_hbm.at[pl.ds(base, bt)], o1_v)
        pltpu.sync_copy(o2_hbm.at[pl.ds(base, bt)], o2_v)
        pltpu.sync_copy(o3_hbm.at[pl.ds(base, bt)], o3_v)
        pltpu.sync_copy(pcum_hbm.at[pl.ds(base, bt)], pcum_v)
        pltpu.sync_copy(valid_hbm.at[pl.ds(base, bt)], valid_v)
        pltpu.sync_copy(nv_hbm, nv_v)

        nv16v = nv_v[...]
        iota16 = lax.iota(jnp.int32, 16)
        cols = [jnp.full((16,), k, jnp.int32) for k in range(4)]

        def group(g, carry_unused):
            r16 = g * 16 + iota16
            v0 = plsc.load_gather(o0_v, [r16])
            v1 = plsc.load_gather(o1_v, [r16])
            v2 = plsc.load_gather(o2_v, [r16])
            v3 = plsc.load_gather(o3_v, [r16])
            plsc.store_scatter(rows_v, [r16, cols[0]], v0)
            plsc.store_scatter(rows_v, [r16, cols[1]], v1)
            plsc.store_scatter(rows_v, [r16, cols[2]], v2)
            plsc.store_scatter(rows_v, [r16, cols[3]], v3)
            pc16 = plsc.load_gather(pcum_v, [r16])
            vm16 = plsc.load_gather(valid_v, [r16]) > 0
            j16 = base + r16
            tgt16 = jnp.where(vm16, pc16, nv16v + (j16 - pc16))
            trow = jnp.full((16,), 0, jnp.int32) + (g // 8)
            tlane = (g % 8) * 16 + iota16
            plsc.store_scatter(tgt_v, [trow, tlane], tgt16)
            return carry_unused

        lax.fori_loop(0, ng, group, 0)

        scatters = [
            pltpu.async_copy(
                rows_v.at[pl.ds(k * 128, 128)],
                out_hbm.at[tgt_v.at[k]], sem)
            for k in range(nch)
        ]
        for cp in scatters:
            cp.wait()

    return sc_kernel(o0, o1, o2, o3, pcum, valid, nv16)


def kernel(cls_logits, cls_regress, proposals):
    b, n, cnum = cls_logits.shape
    logits_t = cls_logits.transpose(2, 0, 1)
    regress_t = cls_regress.transpose(0, 2, 3, 1).reshape(cnum, 4, n)
    props_t = proposals.transpose(0, 2, 1).reshape(4, n)

    info = plsc.get_sparse_core_info()
    nw = info.num_cores * info.num_subcores
    align = nw * 128
    np_rows = ((n + align - 1) // align) * align

    o0, o1, o2, o3, pcum, valid, nv16 = _tc_stage(
        logits_t, regress_t, props_t, np_rows)

    out = _sc_stage(o0, o1, o2, o3, pcum, valid, nv16,
                    np_rows, nw, info.num_cores)
    return lax.stop_gradient(out[:n, :4].reshape(b, n, 4))


# speedup vs baseline: 1.0115x; 1.0115x over previous
import functools
import math

import jax
import jax.numpy as jnp
from jax import lax
from jax.experimental import pallas as pl
from jax.experimental.pallas import tpu as pltpu
from jax.experimental.pallas import tpu_sc as plsc

_STD = (0.1, 0.1, 0.2, 0.2)
_MAX_RATIO = abs(math.log(0.016))


def _tc_body(bn, n, num_classes, lg_ref, rg_ref, pr_ref,
             o0_ref, o1_ref, o2_ref, o3_ref, pcum_ref, valid_ref, nv_ref,
             carry):
    c = pl.program_id(0)

    @pl.when(c == 0)
    def _():
        carry[0] = 0

    lg = lg_ref[...]
    mx = jnp.max(lg, axis=0, keepdims=True)
    rowid = lax.broadcasted_iota(jnp.int32, lg.shape, 0)
    am = jnp.min(jnp.where(lg == mx, rowid, num_classes), axis=0,
                 keepdims=True)

    rg = rg_ref[...]
    sel = lax.broadcasted_iota(jnp.int32, (num_classes, 1, bn), 0) == am
    delta = jnp.sum(jnp.where(sel, rg, 0.0), axis=0)

    dx = delta[0:1] * _STD[0]
    dy = delta[1:2] * _STD[1]
    dw = jnp.clip(delta[2:3] * _STD[2], -_MAX_RATIO, _MAX_RATIO)
    dh = jnp.clip(delta[3:4] * _STD[3], -_MAX_RATIO, _MAX_RATIO)

    pr = pr_ref[...]
    px1, py1, px2, py2 = pr[0:1], pr[1:2], pr[2:3], pr[3:4]
    px = (px1 + px2) * 0.5
    py = (py1 + py2) * 0.5
    pw = px2 - px1
    ph = py2 - py1
    gw = pw * jnp.exp(dw)
    gh = ph * jnp.exp(dh)
    gx = px + pw * dx
    gy = py + ph * dy
    o0 = jnp.clip(gx - gw * 0.5, 0.0, 1.0)
    o1 = jnp.clip(gy - gh * 0.5, 0.0, 1.0)
    o2 = jnp.clip(gx + gw * 0.5, 0.0, 1.0)
    o3 = jnp.clip(gy + gh * 0.5, 0.0, 1.0)

    lane = lax.broadcasted_iota(jnp.int32, (1, bn), 1) + c * bn
    vmask = (jnp.max(pr, axis=0, keepdims=True) > 0.0) & (lane < n)
    zf = jnp.zeros((1, bn), jnp.float32)
    o0 = jnp.where(vmask, o0, zf)
    o1 = jnp.where(vmask, o1, zf)
    o2 = jnp.where(vmask, o2, zf)
    o3 = jnp.where(vmask, o3, zf)

    valid = vmask.astype(jnp.int32)
    x = valid
    s = 1
    while s < bn:
        x = x + jnp.concatenate(
            [jnp.zeros((1, s), jnp.int32), x[:, : bn - s]], axis=1)
        s *= 2
    base = carry[0]
    tot = base + jnp.sum(valid)
    carry[0] = tot

    o0_ref[...] = o0.reshape(bn)
    o1_ref[...] = o1.reshape(bn)
    o2_ref[...] = o2.reshape(bn)
    o3_ref[...] = o3.reshape(bn)
    pcum_ref[...] = ((x - valid) + base).reshape(bn)
    valid_ref[...] = valid.reshape(bn)
    nv_ref[...] = jnp.broadcast_to(tot, (16,)).astype(jnp.int32)


def _tc_stage(logits_t, regress_t, props_t, np_rows):
    cnum = logits_t.shape[0]
    n = logits_t.shape[2]
    bn = 2048
    grid = np_rows // bn
    body = functools.partial(_tc_body, bn, n, cnum)
    o1d = jax.ShapeDtypeStruct((np_rows,), jnp.float32)
    i1d = jax.ShapeDtypeStruct((np_rows,), jnp.int32)
    return pl.pallas_call(
        body,
        grid=(grid,),
        in_specs=[
            pl.BlockSpec((cnum, 1, bn), lambda i: (0, 0, i)),
            pl.BlockSpec((cnum, 4, bn), lambda i: (0, 0, i)),
            pl.BlockSpec((4, bn), lambda i: (0, i)),
        ],
        out_specs=[
            pl.BlockSpec((bn,), lambda i: (i,)),
            pl.BlockSpec((bn,), lambda i: (i,)),
            pl.BlockSpec((bn,), lambda i: (i,)),
            pl.BlockSpec((bn,), lambda i: (i,)),
            pl.BlockSpec((bn,), lambda i: (i,)),
            pl.BlockSpec((bn,), lambda i: (i,)),
            pl.BlockSpec((16,), lambda i: (0,)),
        ],
        out_shape=[o1d, o1d, o1d, o1d, i1d, i1d,
                   jax.ShapeDtypeStruct((16,), jnp.int32)],
        scratch_shapes=[pltpu.SMEM((1,), jnp.int32)],
    )(logits_t, regress_t, props_t)


def _sc_stage(o0, o1, o2, o3, pcum, valid, nv16, np_rows, nw, nc):
    bt = np_rows // nw
    nch = bt // 128
    ng = bt // 16
    mesh = plsc.VectorSubcoreMesh(core_axis_name="c", subcore_axis_name="s")

    @functools.partial(
        pl.kernel,
        mesh=mesh,
        compiler_params=pltpu.CompilerParams(
            needs_layout_passes=False, use_tc_tiling_on_sc=False),
        out_type=jax.ShapeDtypeStruct((np_rows, 8), jnp.float32),
        scratch_types=[
            pltpu.VMEM((bt,), jnp.float32),
            pltpu.VMEM((bt,), jnp.float32),
            pltpu.VMEM((bt,), jnp.float32),
            pltpu.VMEM((bt,), jnp.float32),
            pltpu.VMEM((bt,), jnp.int32),
            pltpu.VMEM((bt,), jnp.int32),
            pltpu.VMEM((bt, 8), jnp.float32),
            pltpu.VMEM((nch, 128), jnp.int32),
            pltpu.VMEM((16,), jnp.int32),
            pltpu.SemaphoreType.DMA,
        ],
    )
    def sc_kernel(o0_hbm, o1_hbm, o2_hbm, o3_hbm, pcum_hbm, valid_hbm,
                  nv_hbm, out_hbm, o0_v, o1_v, o2_v, o3_v, pcum_v, valid_v,
                  rows_v, tgt_v, nv_v, sem):
        wid = lax.axis_index("s") * nc + lax.axis_index("c")
        base = wid * bt
        pltpu.sync_copy(o0_hbm.at[pl.ds(base, bt)], o0_v)
        pltpu.sync_copy(o1_hbm.at[pl.ds(base, bt)], o1_v)
        pltpu.sync_copy(o2_hbm.at[pl.ds(base, bt)], o2_v)
        pltpu.sync_copy(o3_hbm.at[pl.ds(base, bt)], o3_v)
        pltpu.sync_copy(pcum_hbm.at[pl.ds(base, bt)], pcum_v)
        pltpu.sync_copy(valid_hbm.at[pl.ds(base, bt)], valid_v)
        pltpu.sync_copy(nv_hbm, nv_v)

        nv16v = nv_v[...]
        iota16 = lax.iota(jnp.int32, 16)
        cols = [jnp.full((16,), k, jnp.int32) for k in range(4)]

        def group(g, carry_unused):
            r16 = g * 16 + iota16
            v0 = plsc.load_gather(o0_v, [r16])
            v1 = plsc.load_gather(o1_v, [r16])
            v2 = plsc.load_gather(o2_v, [r16])
            v3 = plsc.load_gather(o3_v, [r16])
            plsc.store_scatter(rows_v, [r16, cols[0]], v0)
            plsc.store_scatter(rows_v, [r16, cols[1]], v1)
            plsc.store_scatter(rows_v, [r16, cols[2]], v2)
            plsc.store_scatter(rows_v, [r16, cols[3]], v3)
            pc16 = plsc.load_gather(pcum_v, [r16])
            vm16 = plsc.load_gather(valid_v, [r16]) > 0
            j16 = base + r16
            tgt16 = jnp.where(vm16, pc16, nv16v + (j16 - pc16))
            trow = jnp.full((16,), 0, jnp.int32) + (g // 8)
            tlane = (g % 8) * 16 + iota16
            plsc.store_scatter(tgt_v, [trow, tlane], tgt16)
            return carry_unused

        lax.fori_loop(0, ng, group, 0)

        scatters = [
            pltpu.async_copy(
                rows_v.at[pl.ds(k * 128, 128)],
                out_hbm.at[tgt_v.at[k]], sem)
            for k in range(nch)
        ]
        for cp in scatters:
            cp.wait()

    return sc_kernel(o0, o1, o2, o3, pcum, valid, nv16)


def kernel(cls_logits, cls_regress, proposals):
    b, n, cnum = cls_logits.shape
    logits_t = cls_logits.transpose(2, 0, 1)
    regress_t = cls_regress.transpose(0, 2, 3, 1).reshape(cnum, 4, n)
    props_t = proposals.transpose(0, 2, 1).reshape(4, n)

    info = plsc.get_sparse_core_info()
    nw = info.num_cores * info.num_subcores
    align = nw * 128
    np_rows = ((n + align - 1) // align) * align

    o0, o1, o2, o3, pcum, valid, nv16 = _tc_stage(
        logits_t, regress_t, props_t, np_rows)

    out = _sc_stage(o0, o1, o2, o3, pcum, valid, nv16,
                    np_rows, nw, info.num_cores)
    return lax.stop_gradient(out[:n, :4].reshape(b, n, 4))
